# baseline (speedup 1.0000x reference)
import functools

import jax
import jax.numpy as jnp
from jax import lax
from jax.experimental import pallas as pl
from jax.experimental.pallas import tpu as pltpu
from jax.experimental.pallas import tpu_sc as plsc

_B = 16384
_C = 128
_NC = 2
_NS = 16
_NW = _NC * _NS
_ROWS_PER_W = _B // _NW
_CHUNK = 128
_NCHUNK = _ROWS_PER_W // _CHUNK
_G = _C // 16


def _body(params_hbm, idx_hbm, out_hbm, idx_v, in_v, out_v):
    wid = lax.axis_index("s") * _NC + lax.axis_index("c")
    pltpu.sync_copy(idx_hbm, idx_v)
    idx_g = [idx_v[pl.ds(g * 16, 16)] for g in range(_G)]
    base = wid * _ROWS_PER_W * _C
    for c in range(_NCHUNK):
        off = base + c * _CHUNK * _C
        pltpu.sync_copy(params_hbm.at[pl.ds(off, _CHUNK * _C)], in_v)

        def row_body(r, carry):
            rb = r * _C
            rbv = jnp.full((16,), rb, jnp.int32)
            for g in range(_G):
                vals = plsc.load_gather(in_v, [rbv + idx_g[g]])
                out_v[pl.ds(rb + g * 16, 16)] = vals
            return carry

        lax.fori_loop(0, _CHUNK, row_body, 0)
        pltpu.sync_copy(out_v, out_hbm.at[pl.ds(off, _CHUNK * _C)])


_sc_call = functools.partial(
    pl.kernel,
    out_type=jax.ShapeDtypeStruct((_B * _C,), jnp.float32),
    mesh=plsc.VectorSubcoreMesh(core_axis_name="c", subcore_axis_name="s"),
    scratch_types=[
        pltpu.VMEM((_C,), jnp.int32),
        pltpu.VMEM((_CHUNK * _C,), jnp.float32),
        pltpu.VMEM((_CHUNK * _C,), jnp.float32),
    ],
    compiler_params=pltpu.CompilerParams(needs_layout_passes=False),
)(_body)


@jax.jit
def kernel(parameters, marginal_indices):
    flat = _sc_call(parameters.reshape(-1), marginal_indices.reshape(-1))
    return flat.reshape(_B, _C // 2, 2)


# speedup vs baseline: 1.3710x; 1.3710x over previous
import functools

import jax
import jax.numpy as jnp
from jax import lax
from jax.experimental import pallas as pl
from jax.experimental.pallas import tpu as pltpu
from jax.experimental.pallas import tpu_sc as plsc

_B = 16384
_C = 128
_NC = 2
_NS = 16
_NW = _NC * _NS
_ROWS_PER_W = _B // _NW
_CHUNK = 128
_NCHUNK = _ROWS_PER_W // _CHUNK
_G = _C // 16


def _body(params_hbm, idx_hbm, out_hbm, idx_v, in_v, out_v):
    wid = lax.axis_index("s") * _NC + lax.axis_index("c")
    pltpu.sync_copy(idx_hbm, idx_v)
    lane = jnp.arange(16, dtype=jnp.int32)
    idx_g = [
        plsc.load_gather(idx_v, [g * 8 + (lane >> 1), lane & 1])
        for g in range(_G)
    ]
    row0 = wid * _ROWS_PER_W
    for c in range(_NCHUNK):
        roff = row0 + c * _CHUNK
        pltpu.sync_copy(params_hbm.at[pl.ds(roff, _CHUNK)], in_v)

        def row_body(r, carry):
            rv = jnp.full((16,), r, jnp.int32)
            for g in range(_G):
                vals = plsc.load_gather(in_v, [rv, idx_g[g]])
                plsc.store_scatter(
                    out_v, [rv, g * 8 + (lane >> 1), lane & 1], vals)
            return carry

        lax.fori_loop(0, _CHUNK, row_body, 0)
        pltpu.sync_copy(out_v, out_hbm.at[pl.ds(roff, _CHUNK)])


_sc_call = functools.partial(
    pl.kernel,
    out_type=jax.ShapeDtypeStruct((_B, _C // 2, 2), jnp.float32),
    mesh=plsc.VectorSubcoreMesh(core_axis_name="c", subcore_axis_name="s"),
    scratch_types=[
        pltpu.VMEM((_C // 2, 2), jnp.int32),
        pltpu.VMEM((_CHUNK, _C), jnp.float32),
        pltpu.VMEM((_CHUNK, _C // 2, 2), jnp.float32),
    ],
    compiler_params=pltpu.CompilerParams(
        needs_layout_passes=False, use_tc_tiling_on_sc=False),
)(_body)


@jax.jit
def kernel(parameters, marginal_indices):
    return _sc_call(parameters, marginal_indices)


# speedup vs baseline: 9.9543x; 7.2606x over previous
import functools

import jax
import jax.numpy as jnp
from jax import lax
from jax.experimental import pallas as pl
from jax.experimental.pallas import tpu as pltpu
from jax.experimental.pallas import tpu_sc as plsc

_B = 16384
_C = 128
_NC = 2
_NS = 16
_NW = _NC * _NS
_T = 128
_NT = _B // (_T * _NW)
_G = _C // 16


def _body(params_hbm, idx_hbm, out_hbm, idx_v, ridx_v,
          in_v0, in_v1, st_v0, st_v1, sem_in, sem_out):
    wid = lax.axis_index("s") * _NC + lax.axis_index("c")
    pltpu.sync_copy(idx_hbm, idx_v)

    lane = jnp.arange(16, dtype=jnp.int32)
    rows_g = [g * 16 + lane for g in range(_G)]

    for t in range(_NT):
        tb2 = (wid * _NT + t) * 2
        for g in range(_G):
            cvec = rows_g[g]
            r = (cvec >> 1) * 256 + tb2 + (cvec & 1)
            plsc.store_scatter(ridx_v, [jnp.full((16,), t, jnp.int32), cvec], r)

    in_bufs = [in_v0, in_v1]
    st_bufs = [st_v0, st_v1]

    def start_in(t):
        row0 = (wid * _NT + t) * _T
        return pltpu.async_copy(
            params_hbm.at[pl.ds(row0, _T)], in_bufs[t % 2], sem_in)

    def compute(t):
        in_v = in_bufs[t % 2]
        st_v = st_bufs[t % 2]

        def col_body(c, carry):
            colv = plsc.load_gather(
                idx_v, [jnp.full((16,), c >> 1, jnp.int32),
                        jnp.full((16,), c & 1, jnp.int32)])
            cv = jnp.full((16,), c, jnp.int32)
            for g in range(_G):
                vals = plsc.load_gather(in_v, [rows_g[g], colv])
                plsc.store_scatter(st_v, [cv, rows_g[g]], vals)
            return carry

        lax.fori_loop(0, _C, col_body, 0)

    def start_out(t):
        return pltpu.async_copy(
            st_bufs[t % 2], out_hbm.at[ridx_v.at[t]], sem_out)

    in_descs = [start_in(0)]
    out_descs = []
    for t in range(_NT):
        in_descs[t].wait()
        if t + 1 < _NT:
            in_descs.append(start_in(t + 1))
        compute(t)
        if t >= 2:
            out_descs[t - 2].wait()
        out_descs.append(start_out(t))
    out_descs[_NT - 2].wait()
    out_descs[_NT - 1].wait()


_sc_call = functools.partial(
    pl.kernel,
    out_type=jax.ShapeDtypeStruct((_B, _C), jnp.float32),
    mesh=plsc.VectorSubcoreMesh(core_axis_name="c", subcore_axis_name="s"),
    scratch_types=[
        pltpu.VMEM((_C // 2, 2), jnp.int32),
        pltpu.VMEM((_NT, _C), jnp.int32),
        pltpu.VMEM((_T, _C), jnp.float32),
        pltpu.VMEM((_T, _C), jnp.float32),
        pltpu.VMEM((_T, _C), jnp.float32),
        pltpu.VMEM((_T, _C), jnp.float32),
        pltpu.SemaphoreType.DMA,
        pltpu.SemaphoreType.DMA,
    ],
    compiler_params=pltpu.CompilerParams(
        needs_layout_passes=False, use_tc_tiling_on_sc=False),
)(_body)


@jax.jit
def kernel(parameters, marginal_indices):
    r = _sc_call(parameters, marginal_indices)
    r4 = r.reshape(_C // 2, _B // _T, 2, _T)
    return r4.transpose(1, 3, 0, 2).reshape(_B, _C // 2, 2)


# speedup vs baseline: 17.4838x; 1.7564x over previous
import functools

import jax
import jax.numpy as jnp
from jax import lax
from jax.experimental import pallas as pl
from jax.experimental.pallas import tpu as pltpu
from jax.experimental.pallas import tpu_sc as plsc

_B = 16384
_C = 128
_NC = 2
_NS = 16
_NW = _NC * _NS
_T = 128
_NT = _B // (_T * _NW)
_G = _C // 16


def _body(params_hbm, idx_hbm, out_hbm, idx_v, ridx_v,
          in_v0, in_v1, st_v0, st_v1, sem_in, sem_out):
    wid = lax.axis_index("s") * _NC + lax.axis_index("c")
    pltpu.sync_copy(idx_hbm, idx_v)

    lane = jnp.arange(16, dtype=jnp.int32)
    rows_g = [g * 16 + lane for g in range(_G)]

    for t in range(_NT):
        tb2 = (wid * _NT + t) * 2
        for g in range(_G):
            cvec = rows_g[g]
            r = (cvec >> 1) * 256 + tb2 + (cvec & 1)
            plsc.store_scatter(ridx_v, [jnp.full((16,), t, jnp.int32), cvec], r)

    in_bufs = [in_v0, in_v1]
    st_bufs = [st_v0, st_v1]

    def start_in(t):
        row0 = (wid * _NT + t) * _T
        return pltpu.async_copy(
            params_hbm.at[pl.ds(row0, _T)], in_bufs[t % 2], sem_in)

    idx_g = [
        plsc.load_gather(
            idx_v, [(g * 16 + lane) >> 1, (g * 16 + lane) & 1])
        for g in range(_G)
    ]
    perm = [(lane + k) & 15 for k in range(16)]
    cols_g = rows_g

    def compute(t):
        in_v = in_bufs[t % 2]
        st_v = st_bufs[t % 2]

        def rb_body(rb, carry):
            rb16 = rb * 16
            for cb in range(_G):
                for k in range(16):
                    rowv = rb16 + perm[k]
                    vals = plsc.load_gather(in_v, [rowv, idx_g[cb]])
                    plsc.store_scatter(st_v, [cols_g[cb], rowv], vals)
            return carry

        lax.fori_loop(0, _G, rb_body, 0)

    def start_out(t):
        return pltpu.async_copy(
            st_bufs[t % 2], out_hbm.at[ridx_v.at[t]], sem_out)

    in_descs = [start_in(0)]
    out_descs = []
    for t in range(_NT):
        in_descs[t].wait()
        if t + 1 < _NT:
            in_descs.append(start_in(t + 1))
        compute(t)
        if t >= 2:
            out_descs[t - 2].wait()
        out_descs.append(start_out(t))
    out_descs[_NT - 2].wait()
    out_descs[_NT - 1].wait()


_sc_call = functools.partial(
    pl.kernel,
    out_type=jax.ShapeDtypeStruct((_B, _C), jnp.float32),
    mesh=plsc.VectorSubcoreMesh(core_axis_name="c", subcore_axis_name="s"),
    scratch_types=[
        pltpu.VMEM((_C // 2, 2), jnp.int32),
        pltpu.VMEM((_NT, _C), jnp.int32),
        pltpu.VMEM((_T, _C), jnp.float32),
        pltpu.VMEM((_T, _C), jnp.float32),
        pltpu.VMEM((_T, _C), jnp.float32),
        pltpu.VMEM((_T, _C), jnp.float32),
        pltpu.SemaphoreType.DMA,
        pltpu.SemaphoreType.DMA,
    ],
    compiler_params=pltpu.CompilerParams(
        needs_layout_passes=False, use_tc_tiling_on_sc=False),
)(_body)


@jax.jit
def kernel(parameters, marginal_indices):
    r = _sc_call(parameters, marginal_indices)
    r4 = r.reshape(_C // 2, _B // _T, 2, _T)
    return r4.transpose(1, 3, 0, 2).reshape(_B, _C // 2, 2)


# speedup vs baseline: 18.6308x; 1.0656x over previous
import functools

import jax
import jax.numpy as jnp
from jax import lax
from jax.experimental import pallas as pl
from jax.experimental.pallas import tpu as pltpu
from jax.experimental.pallas import tpu_sc as plsc

_B = 16384
_C = 128
_NC = 2
_NS = 16
_NW = _NC * _NS
_T = 128
_NT = _B // (_T * _NW)
_G = _C // 16


def _body(params_hbm, idx_hbm, out_hbm, idx_v, ridx_v,
          in_v0, in_v1, st_v0, st_v1, sem_in, sem_out):
    wid = lax.axis_index("s") * _NC + lax.axis_index("c")
    pltpu.sync_copy(idx_hbm, idx_v)

    lane = jnp.arange(16, dtype=jnp.int32)
    rows_g = [g * 16 + lane for g in range(_G)]

    for t in range(_NT):
        tb2 = (wid * _NT + t) * 2
        for g in range(_G):
            cvec = rows_g[g]
            r = (cvec >> 1) * 256 + tb2 + (cvec & 1)
            plsc.store_scatter(ridx_v, [jnp.full((16,), t, jnp.int32), cvec], r)

    in_bufs = [in_v0, in_v1]
    st_bufs = [st_v0, st_v1]

    def start_in(t):
        row0 = (wid * _NT + t) * _T
        return pltpu.async_copy(
            params_hbm.at[pl.ds(row0, _T)], in_bufs[t % 2], sem_in)

    idx_g = [
        plsc.load_gather(
            idx_v, [(g * 16 + lane) >> 1, (g * 16 + lane) & 1])
        for g in range(_G)
    ]
    cols_g = rows_g

    def compute(t):
        in_v = in_bufs[t % 2]
        st_v = st_bufs[t % 2]

        def diag_body(m, carry):
            rowv = (m >> 4) * 16 + ((lane + (m & 15)) & 15)
            for cb in range(_G):
                vals = plsc.load_gather(in_v, [rowv, idx_g[cb]])
                plsc.store_scatter(st_v, [cols_g[cb], rowv], vals)
            return carry

        lax.fori_loop(0, _G * 16, diag_body, 0)

    def start_out(t):
        return pltpu.async_copy(
            st_bufs[t % 2], out_hbm.at[ridx_v.at[t]], sem_out)

    in_descs = [start_in(0)]
    out_descs = []
    for t in range(_NT):
        in_descs[t].wait()
        if t + 1 < _NT:
            in_descs.append(start_in(t + 1))
        compute(t)
        if t >= 2:
            out_descs[t - 2].wait()
        out_descs.append(start_out(t))
    out_descs[_NT - 2].wait()
    out_descs[_NT - 1].wait()


_sc_call = functools.partial(
    pl.kernel,
    out_type=jax.ShapeDtypeStruct((_B, _C), jnp.float32),
    mesh=plsc.VectorSubcoreMesh(core_axis_name="c", subcore_axis_name="s"),
    scratch_types=[
        pltpu.VMEM((_C // 2, 2), jnp.int32),
        pltpu.VMEM((_NT, _C), jnp.int32),
        pltpu.VMEM((_T, _C), jnp.float32),
        pltpu.VMEM((_T, _C), jnp.float32),
        pltpu.VMEM((_T, _C), jnp.float32),
        pltpu.VMEM((_T, _C), jnp.float32),
        pltpu.SemaphoreType.DMA,
        pltpu.SemaphoreType.DMA,
    ],
    compiler_params=pltpu.CompilerParams(
        needs_layout_passes=False, use_tc_tiling_on_sc=False),
)(_body)


@jax.jit
def kernel(parameters, marginal_indices):
    r = _sc_call(parameters, marginal_indices)
    r4 = r.reshape(_C // 2, _B // _T, 2, _T)
    return r4.transpose(1, 3, 0, 2).reshape(_B, _C // 2, 2)


# speedup vs baseline: 18.6309x; 1.0000x over previous
import functools

import jax
import jax.numpy as jnp
from jax import lax
from jax.experimental import pallas as pl
from jax.experimental.pallas import tpu as pltpu
from jax.experimental.pallas import tpu_sc as plsc

_B = 16384
_C = 128
_NC = 2
_NS = 16
_NW = _NC * _NS
_T = 128
_NT = _B // (_T * _NW)
_G = _C // 16


def _body(params_hbm, idx_hbm, out_hbm, idx_v, ridx_v,
          in_v0, in_v1, st_v0, st_v1, sem_in, sem_out):
    wid = lax.axis_index("s") * _NC + lax.axis_index("c")
    pltpu.sync_copy(idx_hbm, idx_v)

    lane = jnp.arange(16, dtype=jnp.int32)
    rows_g = [g * 16 + lane for g in range(_G)]

    for t in range(_NT):
        tb2 = (wid * _NT + t) * 2
        for g in range(_G):
            cvec = rows_g[g]
            r = (cvec >> 1) * 256 + tb2 + (cvec & 1)
            plsc.store_scatter(ridx_v, [jnp.full((16,), t, jnp.int32), cvec], r)

    in_bufs = [in_v0, in_v1]
    st_bufs = [st_v0, st_v1]

    def start_in(t):
        row0 = (wid * _NT + t) * _T
        return pltpu.async_copy(
            params_hbm.at[pl.ds(row0, _T)], in_bufs[t % 2], sem_in)

    idx_g = [
        plsc.load_gather(
            idx_v, [(g * 16 + lane) >> 1, (g * 16 + lane) & 1])
        for g in range(_G)
    ]
    cols_g = rows_g

    def compute(t):
        in_v = in_bufs[t % 2]
        st_v = st_bufs[t % 2]

        def diag_body(m, carry):
            rowv = (m >> 4) * 16 + ((lane + (m & 15)) & 15)
            for cb in range(_G):
                vals = plsc.load_gather(in_v, [rowv, idx_g[cb]])
                plsc.store_scatter(st_v, [cols_g[cb], rowv], vals)
            return carry

        lax.fori_loop(0, _G * 16, diag_body, 0)

    def start_out(t):
        return pltpu.async_copy(
            st_bufs[t % 2], out_hbm.at[ridx_v.at[t]], sem_out)

    in_descs = [start_in(0)]
    out_descs = []
    for t in range(_NT):
        in_descs[t].wait()
        if t + 1 < _NT:
            in_descs.append(start_in(t + 1))
        compute(t)
        if t >= 2:
            out_descs[t - 2].wait()
        out_descs.append(start_out(t))
    out_descs[_NT - 2].wait()
    out_descs[_NT - 1].wait()


_sc_call = functools.partial(
    pl.kernel,
    out_type=jax.ShapeDtypeStruct((_B, _C), jnp.float32),
    mesh=plsc.VectorSubcoreMesh(core_axis_name="c", subcore_axis_name="s"),
    scratch_types=[
        pltpu.VMEM((_C // 2, 2), jnp.int32),
        pltpu.VMEM((_NT, _C), jnp.int32),
        pltpu.VMEM((_T, _C), jnp.float32),
        pltpu.VMEM((_T, _C), jnp.float32),
        pltpu.VMEM((_T, _C), jnp.float32),
        pltpu.VMEM((_T, _C), jnp.float32),
        pltpu.SemaphoreType.DMA,
        pltpu.SemaphoreType.DMA,
    ],
    compiler_params=pltpu.CompilerParams(
        needs_layout_passes=False, use_tc_tiling_on_sc=False,
        disable_bounds_checks=True, disable_semaphore_checks=True),
)(_body)


@jax.jit
def kernel(parameters, marginal_indices):
    r = _sc_call(parameters, marginal_indices)
    r4 = r.reshape(_C // 2, _B // _T, 2, _T)
    return r4.transpose(1, 3, 0, 2).reshape(_B, _C // 2, 2)


# speedup vs baseline: 25.5184x; 1.3697x over previous
import functools

import jax
import jax.numpy as jnp
from jax import lax
from jax.experimental import pallas as pl
from jax.experimental.pallas import tpu as pltpu
from jax.experimental.pallas import tpu_sc as plsc

_B = 16384
_C = 128
_NC = 2
_NS = 16
_NW = _NC * _NS
_T = 128
_NT = _B // (_T * _NW)
_G = _C // 16


def _body(params_hbm, idx_hbm, out_hbm, idx_v, ridx_v,
          in_v0, in_v1, st_v0, st_v1, sem_in, sem_out):
    wid = lax.axis_index("s") * _NC + lax.axis_index("c")

    lane = jnp.arange(16, dtype=jnp.int32)
    rows_g = [g * 16 + lane for g in range(_G)]

    in_bufs = [in_v0, in_v1]
    st_bufs = [st_v0, st_v1]

    def start_in(t):
        row0 = (wid * _NT + t) * _T
        return pltpu.async_copy(
            params_hbm.at[pl.ds(row0, _T)], in_bufs[t % 2], sem_in)

    in_descs = [start_in(0)]
    idx_desc = pltpu.async_copy(idx_hbm, idx_v, sem_out)

    for t in range(_NT):
        tb2 = (wid * _NT + t) * 2
        for g in range(_G):
            cvec = rows_g[g]
            r = (cvec >> 1) * 256 + tb2 + (cvec & 1)
            plsc.store_scatter(ridx_v, [jnp.full((16,), t, jnp.int32), cvec], r)

    idx_desc.wait()

    idx_g = [
        plsc.load_gather(
            idx_v, [(g * 16 + lane) >> 1, (g * 16 + lane) & 1])
        for g in range(_G)
    ]
    cols_g = rows_g

    def compute(t):
        in_v = in_bufs[t % 2]
        st_v = st_bufs[t % 2]

        @plsc.parallel_loop(0, _G * 16, 1, unroll=2)
        def diag_body(m):
            rowv = (m >> 4) * 16 + ((lane + (m & 15)) & 15)
            for cb in range(_G):
                vals = plsc.load_gather(in_v, [rowv, idx_g[cb]])
                plsc.store_scatter(st_v, [cols_g[cb], rowv], vals)

    def start_out(t):
        return pltpu.async_copy(
            st_bufs[t % 2], out_hbm.at[ridx_v.at[t]], sem_out)

    out_descs = []
    for t in range(_NT):
        in_descs[t].wait()
        if t + 1 < _NT:
            in_descs.append(start_in(t + 1))
        compute(t)
        if t >= 2:
            out_descs[t - 2].wait()
        out_descs.append(start_out(t))
    out_descs[_NT - 2].wait()
    out_descs[_NT - 1].wait()


_sc_call = functools.partial(
    pl.kernel,
    out_type=jax.ShapeDtypeStruct((_B, _C), jnp.float32),
    mesh=plsc.VectorSubcoreMesh(core_axis_name="c", subcore_axis_name="s"),
    scratch_types=[
        pltpu.VMEM((_C // 2, 2), jnp.int32),
        pltpu.VMEM((_NT, _C), jnp.int32),
        pltpu.VMEM((_T, _C), jnp.float32),
        pltpu.VMEM((_T, _C), jnp.float32),
        pltpu.VMEM((_T, _C), jnp.float32),
        pltpu.VMEM((_T, _C), jnp.float32),
        pltpu.SemaphoreType.DMA,
        pltpu.SemaphoreType.DMA,
    ],
    compiler_params=pltpu.CompilerParams(
        needs_layout_passes=False, use_tc_tiling_on_sc=False,
        disable_bounds_checks=True, disable_semaphore_checks=True),
)(_body)


@jax.jit
def kernel(parameters, marginal_indices):
    r = _sc_call(parameters, marginal_indices)
    r4 = r.reshape(_C // 2, _B // _T, 2, _T)
    return r4.transpose(1, 3, 0, 2).reshape(_B, _C // 2, 2)
